# baseline (speedup 1.0000x reference)
import functools

import jax
import jax.numpy as jnp
from jax import lax
from jax.experimental import pallas as pl
from jax.experimental.pallas import tpu as pltpu
from jax.experimental.pallas import tpu_sc as plsc

D = 128
B = 16384
NC = 2
NS = 16
NW = NC * NS

B_SC = 4096
B_TC = B - B_SC
B_PER_W = B_SC // NW
CHUNK = 128
N_CHUNKS = B_PER_W // CHUNK

BLK = 1024
NB_TC = B_TC // BLK
K_HI = 392
K_LO = 256


def _sc_body(te_hbm, t_hbm, out_hbm, idx_v, rows_v, gsem, ssem):
    wid = lax.axis_index("s") * NC + lax.axis_index("c")
    base = wid * B_PER_W
    pltpu.sync_copy(t_hbm.at[pl.ds(base, B_PER_W)], idx_v)

    def gather(j):
        return pltpu.async_copy(
            te_hbm.at[idx_v.at[pl.ds(j * CHUNK, CHUNK)]],
            rows_v.at[pl.ds(j * CHUNK, CHUNK)],
            gsem,
        )

    def scatter(j):
        return pltpu.async_copy(
            rows_v.at[pl.ds(j * CHUNK, CHUNK)],
            out_hbm.at[pl.ds(base + j * CHUNK, CHUNK)],
            ssem,
        )

    gathers = {j: gather(j) for j in range(min(2, N_CHUNKS))}
    scatters = []
    for j in range(N_CHUNKS):
        gathers[j].wait()
        if j + 2 < N_CHUNKS:
            gathers[j + 2] = gather(j + 2)
        scatters.append(scatter(j))
    for s in scatters:
        s.wait()


def _tc_body(tcol_ref, thi_ref, thip_ref, tlo_ref, tlop_ref, out_ref):
    tcol = tcol_ref[:, :]
    hi = lax.shift_right_logical(tcol, 8)
    lo = lax.bitwise_and(tcol, 255)
    ih = lax.broadcasted_iota(jnp.int32, (BLK, K_HI), 1)
    il = lax.broadcasted_iota(jnp.int32, (BLK, K_LO), 1)
    oh_hi = (hi == ih).astype(jnp.float32)
    oh_lo = (lo == il).astype(jnp.float32)
    prec = lax.Precision.HIGHEST
    a = jnp.dot(oh_hi, thi_ref[:, :], precision=prec)
    ap = jnp.dot(oh_hi, thip_ref[:, :], precision=prec)
    b = jnp.dot(oh_lo, tlo_ref[:, :], precision=prec)
    bp = jnp.dot(oh_lo, tlop_ref[:, :], precision=prec)
    out_ref[:, :] = a * b + ap * bp


@jax.jit
def kernel(te, t):
    mesh = plsc.VectorSubcoreMesh(core_axis_name="c", subcore_axis_name="s")
    sc_run = functools.partial(
        pl.kernel,
        out_type=jax.ShapeDtypeStruct((B_SC, D), jnp.float32),
        mesh=mesh,
        scratch_types=[
            pltpu.VMEM((B_PER_W,), jnp.int32),
            pltpu.VMEM((B_PER_W, D), jnp.float32),
            pltpu.SemaphoreType.DMA,
            pltpu.SemaphoreType.DMA,
        ],
    )(_sc_body)
    sc_part = sc_run(te, t)

    hi_tab = jnp.pad(te[::256], ((0, K_HI - 391), (0, 0)))
    swap = hi_tab.reshape(K_HI, D // 2, 2)[:, :, ::-1].reshape(K_HI, D)
    sgn = jnp.tile(jnp.array([1.0, -1.0], jnp.float32), D // 2)
    thip = swap * sgn
    lo_tab = te[:K_LO]
    tlo = jnp.repeat(lo_tab[:, 1::2], 2, axis=1)
    tlop = jnp.repeat(lo_tab[:, 0::2], 2, axis=1)

    t_col = t[B_SC:].reshape(B_TC, 1)
    nb_sc = B_SC // BLK
    tc_full = pl.pallas_call(
        _tc_body,
        grid=(NB_TC,),
        in_specs=[
            pl.BlockSpec((BLK, 1), lambda i: (i, 0)),
            pl.BlockSpec((K_HI, D), lambda i: (0, 0)),
            pl.BlockSpec((K_HI, D), lambda i: (0, 0)),
            pl.BlockSpec((K_LO, D), lambda i: (0, 0)),
            pl.BlockSpec((K_LO, D), lambda i: (0, 0)),
        ],
        out_specs=pl.BlockSpec((BLK, D), lambda i: (i + nb_sc, 0)),
        out_shape=jax.ShapeDtypeStruct((B, D), jnp.float32),
    )(t_col, hi_tab, thip, tlo, tlop)

    return lax.dynamic_update_slice(tc_full, sc_part, (0, 0))


# speedup vs baseline: 4.3222x; 4.3222x over previous
import functools

import jax
import jax.numpy as jnp
from jax import lax
from jax.experimental import pallas as pl
from jax.experimental.pallas import tpu as pltpu
from jax.experimental.pallas import tpu_sc as plsc

D = 128
B = 16384
NC = 2
NS = 16
NW = NC * NS
B_PER_W = B // NW

CHUNK_SIZES = (32, 96, 128, 128, 96, 32)
CHUNK_OFFS = (0, 32, 128, 256, 384, 480)
N_CHUNKS = len(CHUNK_SIZES)


def _gather_body(te_hbm, t_hbm, out_hbm, idx_v, rows_v, gsem, ssem):
    wid = lax.axis_index("s") * NC + lax.axis_index("c")
    base = wid * B_PER_W
    pltpu.sync_copy(t_hbm.at[pl.ds(base, B_PER_W)], idx_v)

    def gather(j):
        return pltpu.async_copy(
            te_hbm.at[idx_v.at[pl.ds(CHUNK_OFFS[j], CHUNK_SIZES[j])]],
            rows_v.at[pl.ds(CHUNK_OFFS[j], CHUNK_SIZES[j])],
            gsem,
        )

    def scatter(j):
        return pltpu.async_copy(
            rows_v.at[pl.ds(CHUNK_OFFS[j], CHUNK_SIZES[j])],
            out_hbm.at[pl.ds(base + CHUNK_OFFS[j], CHUNK_SIZES[j])],
            ssem,
        )

    gathers = {j: gather(j) for j in range(min(2, N_CHUNKS))}
    scatters = []
    for j in range(N_CHUNKS):
        gathers[j].wait()
        if j + 2 < N_CHUNKS:
            gathers[j + 2] = gather(j + 2)
        scatters.append(scatter(j))
    for s in scatters:
        s.wait()


@jax.jit
def kernel(te, t):
    mesh = plsc.VectorSubcoreMesh(core_axis_name="c", subcore_axis_name="s")
    run = functools.partial(
        pl.kernel,
        out_type=jax.ShapeDtypeStruct((B, D), jnp.float32),
        mesh=mesh,
        scratch_types=[
            pltpu.VMEM((B_PER_W,), jnp.int32),
            pltpu.VMEM((B_PER_W, D), jnp.float32),
            pltpu.SemaphoreType.DMA,
            pltpu.SemaphoreType.DMA,
        ],
    )(_gather_body)
    return run(te, t)
